# Initial kernel scaffold; baseline (speedup 1.0000x reference)
import jax
import jax.numpy as jnp
from jax.experimental import pallas as pl


def kernel(inp, embed_table, W1, b1, W2, b2):
    raise NotImplementedError("write your pallas kernel here")


# speedup vs baseline: 2.7114x; 2.7114x over previous
import functools

import jax
import jax.numpy as jnp
from jax import lax
from jax.experimental import pallas as pl
from jax.experimental.pallas import tpu as pltpu
from jax.experimental.pallas import tpu_sc as plsc

B = 16384
D = 208
NC = 2
NS = 16
NW = NC * NS
BPW = B // NW
CH = 128
NCH = BPW // CH
L = 16


def _precompute_body(e_ref, w1_ref, b1_ref, p1_ref, p2_ref):
    e = e_ref[...]
    p1 = jnp.dot(e, w1_ref[0:256, :], preferred_element_type=jnp.float32)
    p1 = p1 + b1_ref[...]
    p2 = jnp.dot(e, w1_ref[256:512, :], preferred_element_type=jnp.float32)
    pad = jnp.zeros((256, D - 200), jnp.float32)
    p1_ref[...] = jnp.concatenate([p1, pad], axis=1)
    p2_ref[...] = jnp.concatenate([p2, pad], axis=1)


def _precompute(embed_table, w1, b1_row):
    return pl.pallas_call(
        _precompute_body,
        out_shape=(
            jax.ShapeDtypeStruct((256, D), jnp.float32),
            jax.ShapeDtypeStruct((256, D), jnp.float32),
        ),
    )(embed_table, w1, b1_row)


def _sc_gather_add(p1_hbm, p2_hbm, idx0_hbm, idx1_hbm, z_hbm,
                   idx0_v, idx1_v, r1, r2, sem1, sem2):
    wid = lax.axis_index("s") * NC + lax.axis_index("c")
    base = wid * BPW
    pltpu.sync_copy(idx0_hbm.at[pl.ds(base, BPW)], idx0_v)
    pltpu.sync_copy(idx1_hbm.at[pl.ds(base, BPW)], idx1_v)
    for c in range(NCH):
        cp1 = pltpu.async_copy(p1_hbm.at[idx0_v.at[pl.ds(c * CH, CH)]], r1, sem1)
        cp2 = pltpu.async_copy(p2_hbm.at[idx1_v.at[pl.ds(c * CH, CH)]], r2, sem2)
        cp1.wait()
        cp2.wait()

        def add_row(i, carry):
            for j in range(D // L):
                sl = pl.ds(j * L, L)
                r1[i, sl] = r1[i, sl] + r2[i, sl]
            return carry

        lax.fori_loop(0, CH, add_row, 0)
        pltpu.sync_copy(r1, z_hbm.at[pl.ds(base + c * CH, CH)])


_sc_gather_add_call = functools.partial(
    pl.kernel,
    out_type=jax.ShapeDtypeStruct((B, D), jnp.float32),
    mesh=plsc.VectorSubcoreMesh(core_axis_name="c", subcore_axis_name="s"),
    compiler_params=pltpu.CompilerParams(use_tc_tiling_on_sc=False),
    scratch_types=[
        pltpu.VMEM((BPW,), jnp.int32),
        pltpu.VMEM((BPW,), jnp.int32),
        pltpu.VMEM((CH, D), jnp.float32),
        pltpu.VMEM((CH, D), jnp.float32),
        pltpu.SemaphoreType.DMA,
        pltpu.SemaphoreType.DMA,
    ],
)(_sc_gather_add)


BT = 2048


def _mlp_body(z_ref, w2_ref, b2_ref, o_ref):
    a1 = jnp.tanh(z_ref[...])
    z2 = jnp.dot(a1, w2_ref[...], preferred_element_type=jnp.float32)
    o_ref[...] = jnp.tanh(z2 + b2_ref[0])


def _mlp(z, w2_pad, b2):
    return pl.pallas_call(
        _mlp_body,
        grid=(B // BT,),
        in_specs=[
            pl.BlockSpec((BT, D), lambda i: (i, 0)),
            pl.BlockSpec((D, 1), lambda i: (0, 0)),
            pl.BlockSpec(memory_space=pltpu.SMEM),
        ],
        out_specs=pl.BlockSpec((BT, 1), lambda i: (i, 0)),
        out_shape=jax.ShapeDtypeStruct((B, 1), jnp.float32),
    )(z, w2_pad, b2)


def kernel(inp, embed_table, W1, b1, W2, b2):
    idx = inp.astype(jnp.int32)
    idx0 = idx[:, 0]
    idx1 = idx[:, 1]
    p1, p2 = _precompute(embed_table, W1, b1.reshape(1, 200))
    z = _sc_gather_add_call(p1, p2, idx0, idx1)
    w2_pad = jnp.pad(W2, ((0, D - 200), (0, 0)))
    return _mlp(z, w2_pad, b2.astype(jnp.float32))


# speedup vs baseline: 3.8364x; 1.4149x over previous
import functools

import jax
import jax.numpy as jnp
from jax import lax
from jax.experimental import pallas as pl
from jax.experimental.pallas import tpu as pltpu
from jax.experimental.pallas import tpu_sc as plsc

B = 16384
D = 208
NC = 2
NS = 16
NW = NC * NS
BPW = B // NW
CH = 128
NCH = BPW // CH
L = 16
NJ = D // L


def _tanh16(x):
    e = jnp.exp(x + x)
    return 1.0 - 2.0 / (e + 1.0)


def _precompute_body(e_ref, w1_ref, b1_ref, p1_ref, p2_ref):
    e = e_ref[...]
    p1 = jnp.dot(e, w1_ref[0:256, :], preferred_element_type=jnp.float32)
    p1 = p1 + b1_ref[...]
    p2 = jnp.dot(e, w1_ref[256:512, :], preferred_element_type=jnp.float32)
    pad = jnp.zeros((256, D - 200), jnp.float32)
    p1_ref[...] = jnp.concatenate([p1, pad], axis=1)
    p2_ref[...] = jnp.concatenate([p2, pad], axis=1)


def _precompute(embed_table, w1, b1_row):
    return pl.pallas_call(
        _precompute_body,
        out_shape=(
            jax.ShapeDtypeStruct((256, D), jnp.float32),
            jax.ShapeDtypeStruct((256, D), jnp.float32),
        ),
    )(embed_table, w1, b1_row)


def _sc_fused(p1_hbm, p2_hbm, idx0_hbm, idx1_hbm, w2_hbm, b2_hbm, out_hbm,
              idx0_v, idx1_v, r1, r2, w2_v, b2_v, out_v, sem1, sem2):
    wid = lax.axis_index("s") * NC + lax.axis_index("c")
    base = wid * BPW
    pltpu.sync_copy(idx0_hbm.at[pl.ds(base, BPW)], idx0_v)
    pltpu.sync_copy(idx1_hbm.at[pl.ds(base, BPW)], idx1_v)
    pltpu.sync_copy(w2_hbm, w2_v)
    pltpu.sync_copy(b2_hbm, b2_v)
    w2r = [w2_v[pl.ds(j * L, L)] for j in range(NJ)]
    lane = lax.iota(jnp.int32, L)
    last_lane = lane == (L - 1)

    for c in range(NCH):
        cp1 = pltpu.async_copy(p1_hbm.at[idx0_v.at[pl.ds(c * CH, CH)]], r1, sem1)
        cp2 = pltpu.async_copy(p2_hbm.at[idx1_v.at[pl.ds(c * CH, CH)]], r2, sem2)
        cp1.wait()
        cp2.wait()

        def row_body(i, carry):
            acc = jnp.zeros((L,), jnp.float32)
            for j in range(NJ):
                sl = pl.ds(j * L, L)
                a1 = _tanh16(r1[i, sl] + r2[i, sl])
                acc = acc + a1 * w2r[j]
            total = plsc.cumsum(acc)
            plsc.store_scatter(out_v, [jnp.full((L,), c * CH + i, jnp.int32)],
                               total, mask=last_lane)
            return carry

        lax.fori_loop(0, CH, row_body, 0)

    b2r = b2_v[...]
    for g in range(BPW // L):
        sl = pl.ds(g * L, L)
        out_v[sl] = _tanh16(out_v[sl] + b2r)
    pltpu.sync_copy(out_v, out_hbm.at[pl.ds(base, BPW)])


_sc_fused_call = functools.partial(
    pl.kernel,
    out_type=jax.ShapeDtypeStruct((B,), jnp.float32),
    mesh=plsc.VectorSubcoreMesh(core_axis_name="c", subcore_axis_name="s"),
    compiler_params=pltpu.CompilerParams(use_tc_tiling_on_sc=False,
                                         needs_layout_passes=False),
    scratch_types=[
        pltpu.VMEM((BPW,), jnp.int32),
        pltpu.VMEM((BPW,), jnp.int32),
        pltpu.VMEM((CH, D), jnp.float32),
        pltpu.VMEM((CH, D), jnp.float32),
        pltpu.VMEM((D,), jnp.float32),
        pltpu.VMEM((L,), jnp.float32),
        pltpu.VMEM((BPW,), jnp.float32),
        pltpu.SemaphoreType.DMA,
        pltpu.SemaphoreType.DMA,
    ],
)(_sc_fused)


def kernel(inp, embed_table, W1, b1, W2, b2):
    idx = inp.astype(jnp.int32)
    idx0 = idx[:, 0]
    idx1 = idx[:, 1]
    p1, p2 = _precompute(embed_table, W1, b1.reshape(1, 200))
    w2_pad = jnp.pad(W2[:, 0], (0, D - 200))
    b2_vec = jnp.broadcast_to(b2.astype(jnp.float32), (L,))
    out = _sc_fused_call(p1, p2, idx0, idx1, w2_pad, b2_vec)
    return out.reshape(B, 1)
